# baseline (device time: 29726 ns/iter reference)
import jax
import jax.numpy as jnp
from jax import lax
from jax.experimental import pallas as pl
from jax.experimental.pallas import tpu as pltpu

NCHUNK = 4


def kernel(x, assign, W1, W2):
    t, d = x.shape
    n_exp, _, f = W1.shape
    tc = t // NCHUNK
    assign2d = assign.reshape(t, 1)

    def body(x_ref, a_ref, w1_ref, w2_ref, out_ref,
             xf_ref, w1f_ref, w2f_ref,
             xs_ref, xr_ref, as_ref, ar_ref, ys_ref, yr_ref,
             w1b_ref, w2b_ref,
             local_sems, x_send_sems, x_recv_sems, a_sems,
             y_send_sems, y_recv_sems):
        my_x = lax.axis_index("x")
        my_y = lax.axis_index("y")
        my_z = lax.axis_index("z")
        peer = (my_x, my_y, 1 - my_z)

        cp_x = pltpu.make_async_copy(x_ref, xf_ref, local_sems.at[0])
        cp_x.start()
        cp_w1 = pltpu.make_async_copy(w1_ref, w1f_ref, local_sems.at[1])
        cp_w1.start()
        cp_w2 = pltpu.make_async_copy(w2_ref, w2f_ref, local_sems.at[2])
        cp_w2.start()

        barrier = pltpu.get_barrier_semaphore()
        pl.semaphore_signal(barrier, inc=1, device_id=peer,
                            device_id_type=pl.DeviceIdType.MESH)
        pl.semaphore_wait(barrier, 1)

        cp_x.wait()
        xs_ref[...] = xf_ref[...].astype(jnp.bfloat16)
        as_ref[...] = a_ref[...]

        chunk = lambda ref, i: ref.at[pl.ds(i * tc, tc), :]
        x_rdmas = []
        for i in range(NCHUNK):
            r = pltpu.make_async_remote_copy(
                src_ref=chunk(xs_ref, i), dst_ref=chunk(xr_ref, i),
                send_sem=x_send_sems.at[i], recv_sem=x_recv_sems.at[i],
                device_id=peer, device_id_type=pl.DeviceIdType.MESH)
            r.start()
            x_rdmas.append(r)
        rdma_a = pltpu.make_async_remote_copy(
            src_ref=as_ref, dst_ref=ar_ref,
            send_sem=a_sems.at[0], recv_sem=a_sems.at[1],
            device_id=peer, device_id_type=pl.DeviceIdType.MESH)
        rdma_a.start()

        cp_w1.wait()
        cp_w2.wait()
        for el in range(n_exp):
            w1b_ref[el] = w1f_ref[el].astype(jnp.bfloat16)
            w2b_ref[el] = w2f_ref[el].astype(jnp.bfloat16)

        e_base = 2 * my_z

        def ffn(x_blk, a_blk):
            m = x_blk.shape[0]
            acc = jnp.zeros((m, d), jnp.float32)
            for el in range(n_exp):
                mask = a_blk == (e_base + el)
                xm = jnp.where(mask, x_blk, jnp.bfloat16(0))
                h = jnp.maximum(
                    jnp.dot(xm, w1b_ref[el],
                            preferred_element_type=jnp.float32),
                    0.0,
                )
                acc = acc + jnp.dot(
                    h.astype(jnp.bfloat16), w2b_ref[el],
                    preferred_element_type=jnp.float32)
            return acc

        out_ref[...] = ffn(xs_ref[...], a_ref[...])

        rdma_a.wait()
        y_rdmas = []
        for i in range(NCHUNK):
            x_rdmas[i].wait()
            sl = pl.ds(i * tc, tc)
            ys_ref[sl, :] = ffn(xr_ref[sl, :], ar_ref[sl, :]).astype(
                jnp.bfloat16)
            r = pltpu.make_async_remote_copy(
                src_ref=chunk(ys_ref, i), dst_ref=chunk(yr_ref, i),
                send_sem=y_send_sems.at[i], recv_sem=y_recv_sems.at[i],
                device_id=peer, device_id_type=pl.DeviceIdType.MESH)
            r.start()
            y_rdmas.append(r)

        for r in y_rdmas:
            r.wait()
        out_ref[...] = out_ref[...] + yr_ref[...].astype(jnp.float32)

    return pl.pallas_call(
        body,
        out_shape=jax.ShapeDtypeStruct((t, d), jnp.float32),
        in_specs=[
            pl.BlockSpec(memory_space=pl.ANY),
            pl.BlockSpec(memory_space=pltpu.VMEM),
            pl.BlockSpec(memory_space=pl.ANY),
            pl.BlockSpec(memory_space=pl.ANY),
        ],
        out_specs=pl.BlockSpec(memory_space=pltpu.VMEM),
        scratch_shapes=[
            pltpu.VMEM((t, d), jnp.float32),
            pltpu.VMEM((n_exp, d, f), jnp.float32),
            pltpu.VMEM((n_exp, f, d), jnp.float32),
            pltpu.VMEM((t, d), jnp.bfloat16),
            pltpu.VMEM((t, d), jnp.bfloat16),
            pltpu.VMEM((t, 1), jnp.int32),
            pltpu.VMEM((t, 1), jnp.int32),
            pltpu.VMEM((t, d), jnp.bfloat16),
            pltpu.VMEM((t, d), jnp.bfloat16),
            pltpu.VMEM((n_exp, d, f), jnp.bfloat16),
            pltpu.VMEM((n_exp, f, d), jnp.bfloat16),
            pltpu.SemaphoreType.DMA((3,)),
            pltpu.SemaphoreType.DMA((NCHUNK,)),
            pltpu.SemaphoreType.DMA((NCHUNK,)),
            pltpu.SemaphoreType.DMA((2,)),
            pltpu.SemaphoreType.DMA((NCHUNK,)),
            pltpu.SemaphoreType.DMA((NCHUNK,)),
        ],
        compiler_params=pltpu.CompilerParams(collective_id=0),
    )(x, assign2d, W1, W2)


# device time: 27887 ns/iter; 1.0659x vs baseline; 1.0659x over previous
import jax
import jax.numpy as jnp
from jax import lax
from jax.experimental import pallas as pl
from jax.experimental.pallas import tpu as pltpu

NCHUNK = 4


def kernel(x, assign, W1, W2):
    t, d = x.shape
    n_exp, _, f = W1.shape
    tc = t // NCHUNK
    assign2d = assign.reshape(t, 1)

    def body(x_ref, a_ref, w1_ref, w2_ref, out_ref,
             xf_ref, w1f_ref, w2f_ref,
             xs_ref, xr_ref, as_ref, ar_ref, ys_ref, yr_ref,
             w1b_ref, w2b_ref,
             local_sems, x_send_sems, x_recv_sems, a_sems,
             y_send_sems, y_recv_sems):
        my_x = lax.axis_index("x")
        my_y = lax.axis_index("y")
        my_z = lax.axis_index("z")
        peer = (my_x, my_y, 1 - my_z)

        cp_x = pltpu.make_async_copy(x_ref, xf_ref, local_sems.at[0])
        cp_x.start()
        cp_w1 = pltpu.make_async_copy(w1_ref, w1f_ref, local_sems.at[1])
        cp_w1.start()
        cp_w2 = pltpu.make_async_copy(w2_ref, w2f_ref, local_sems.at[2])
        cp_w2.start()

        barrier = pltpu.get_barrier_semaphore()
        pl.semaphore_signal(barrier, inc=1, device_id=peer,
                            device_id_type=pl.DeviceIdType.MESH)
        pl.semaphore_wait(barrier, 1)

        cp_x.wait()
        xs_ref[...] = xf_ref[...].astype(jnp.bfloat16)
        as_ref[...] = a_ref[...]

        chunk = lambda ref, i: ref.at[pl.ds(i * tc, tc), :]
        x_rdmas = []
        for i in range(NCHUNK):
            r = pltpu.make_async_remote_copy(
                src_ref=chunk(xs_ref, i), dst_ref=chunk(xr_ref, i),
                send_sem=x_send_sems.at[i], recv_sem=x_recv_sems.at[i],
                device_id=peer, device_id_type=pl.DeviceIdType.MESH)
            r.start()
            x_rdmas.append(r)
        rdma_a = pltpu.make_async_remote_copy(
            src_ref=as_ref, dst_ref=ar_ref,
            send_sem=a_sems.at[0], recv_sem=a_sems.at[1],
            device_id=peer, device_id_type=pl.DeviceIdType.MESH)
        rdma_a.start()

        cp_w1.wait()
        cp_w2.wait()
        for el in range(n_exp):
            w1b_ref[el] = w1f_ref[el].astype(jnp.bfloat16)
            w2b_ref[el] = w2f_ref[el].astype(jnp.bfloat16)

        e_base = 2 * my_z

        def ffn(x_blk, a_blk):
            m = x_blk.shape[0]
            acc = jnp.zeros((m, d), jnp.float32)
            for el in range(n_exp):
                mask = a_blk == (e_base + el)
                xm = jnp.where(mask, x_blk, jnp.bfloat16(0))
                h = jnp.maximum(
                    jnp.dot(xm, w1b_ref[el],
                            preferred_element_type=jnp.float32),
                    0.0,
                )
                acc = acc + jnp.dot(
                    h.astype(jnp.bfloat16), w2b_ref[el],
                    preferred_element_type=jnp.float32)
            return acc

        out_ref[...] = ffn(xs_ref[...], a_ref[...])

        rdma_a.wait()
        y_rdmas = []
        for i in range(NCHUNK):
            x_rdmas[i].wait()
            sl = pl.ds(i * tc, tc)
            ys_ref[sl, :] = ffn(xr_ref[sl, :], ar_ref[sl, :]).astype(
                jnp.bfloat16)
            r = pltpu.make_async_remote_copy(
                src_ref=chunk(ys_ref, i), dst_ref=chunk(yr_ref, i),
                send_sem=y_send_sems.at[i], recv_sem=y_recv_sems.at[i],
                device_id=peer, device_id_type=pl.DeviceIdType.MESH)
            r.start()
            y_rdmas.append(r)

        for r in y_rdmas:
            r.wait()
        out_ref[...] = out_ref[...] + yr_ref[...].astype(jnp.float32)

    return pl.pallas_call(
        body,
        out_shape=jax.ShapeDtypeStruct((t, d), jnp.float32),
        in_specs=[
            pl.BlockSpec(memory_space=pltpu.MemorySpace.HBM),
            pl.BlockSpec(memory_space=pltpu.VMEM),
            pl.BlockSpec(memory_space=pltpu.MemorySpace.HBM),
            pl.BlockSpec(memory_space=pltpu.MemorySpace.HBM),
        ],
        out_specs=pl.BlockSpec(memory_space=pltpu.VMEM),
        scratch_shapes=[
            pltpu.VMEM((t, d), jnp.float32),
            pltpu.VMEM((n_exp, d, f), jnp.float32),
            pltpu.VMEM((n_exp, f, d), jnp.float32),
            pltpu.VMEM((t, d), jnp.bfloat16),
            pltpu.VMEM((t, d), jnp.bfloat16),
            pltpu.VMEM((t, 1), jnp.int32),
            pltpu.VMEM((t, 1), jnp.int32),
            pltpu.VMEM((t, d), jnp.bfloat16),
            pltpu.VMEM((t, d), jnp.bfloat16),
            pltpu.VMEM((n_exp, d, f), jnp.bfloat16),
            pltpu.VMEM((n_exp, f, d), jnp.bfloat16),
            pltpu.SemaphoreType.DMA((3,)),
            pltpu.SemaphoreType.DMA((NCHUNK,)),
            pltpu.SemaphoreType.DMA((NCHUNK,)),
            pltpu.SemaphoreType.DMA((2,)),
            pltpu.SemaphoreType.DMA((NCHUNK,)),
            pltpu.SemaphoreType.DMA((NCHUNK,)),
        ],
        compiler_params=pltpu.CompilerParams(collective_id=0),
    )(x, assign2d, W1, W2)
